# baseline (speedup 1.0000x reference)
import functools

import jax
import jax.numpy as jnp
from jax import lax
from jax.experimental import pallas as pl
from jax.experimental.pallas import tpu as pltpu
from jax.experimental.pallas import tpu_sc as plsc

_B = 16384
_D = 64
_L = 16


def _build():
    info = plsc.get_sparse_core_info()
    nc, ns = info.num_cores, info.num_subcores
    nw = nc * ns
    bpw = _B // nw
    ch = 128
    nch = bpw // ch

    mesh = plsc.VectorSubcoreMesh(core_axis_name="c", subcore_axis_name="s")

    @functools.partial(
        pl.kernel,
        mesh=mesh,
        out_type=jax.ShapeDtypeStruct((nw, _L), jnp.float32),
        scratch_types=[
            pltpu.VMEM((nch, ch), jnp.int32),
            pltpu.VMEM((bpw, _D), jnp.float32),
            pltpu.VMEM((bpw, _D), jnp.float32),
            pltpu.VMEM((_L,), jnp.float32),
            pltpu.SemaphoreType.DMA,
        ],
        compiler_params=pltpu.CompilerParams(use_tc_tiling_on_sc=False),
    )
    def k(feat_hbm, lab_hbm, cent_hbm, out_hbm, idx_v, feat_v, rows_v,
          acc_v, sem):
        wid = lax.axis_index("s") * nc + lax.axis_index("c")
        base = wid * bpw

        for c in range(nch):
            pltpu.sync_copy(lab_hbm.at[pl.ds(base + c * ch, ch)], idx_v.at[c])

        copies = [pltpu.async_copy(feat_hbm.at[pl.ds(base, bpw)], feat_v, sem)]
        for c in range(nch):
            copies.append(
                pltpu.async_copy(
                    cent_hbm.at[idx_v.at[c]],
                    rows_v.at[pl.ds(c * ch, ch)],
                    sem,
                )
            )
        for cp in copies:
            cp.wait()

        def body(i, acc):
            for j in range(_D // _L):
                f = feat_v[i, pl.ds(j * _L, _L)]
                cvec = rows_v[i, pl.ds(j * _L, _L)]
                d = f - cvec
                acc = acc + d * d
            return acc

        acc = lax.fori_loop(0, bpw, body, jnp.zeros((_L,), jnp.float32))
        acc_v[...] = acc
        pltpu.sync_copy(acc_v, out_hbm.at[wid])

    return k


_K = _build()


@jax.jit
def kernel(features, labels, centers):
    partials = _K(features, labels.astype(jnp.int32), centers)
    return jnp.sum(partials) / features.shape[0]


# speedup vs baseline: 1.3089x; 1.3089x over previous
import functools

import jax
import jax.numpy as jnp
from jax import lax
from jax.experimental import pallas as pl
from jax.experimental.pallas import tpu as pltpu
from jax.experimental.pallas import tpu_sc as plsc

_B = 16384
_D = 64
_L = 16


def _build():
    info = plsc.get_sparse_core_info()
    nc, ns = info.num_cores, info.num_subcores
    nw = nc * ns
    bpw = _B // nw
    fch = bpw // 2

    mesh = plsc.VectorSubcoreMesh(core_axis_name="c", subcore_axis_name="s")

    @functools.partial(
        pl.kernel,
        mesh=mesh,
        out_type=jax.ShapeDtypeStruct((nw * _L,), jnp.float32),
        scratch_types=[
            pltpu.VMEM((bpw,), jnp.int32),
            pltpu.VMEM((fch, _D), jnp.float32),
            pltpu.VMEM((bpw, _D), jnp.float32),
            pltpu.VMEM((_L,), jnp.float32),
            pltpu.SemaphoreType.DMA,
            pltpu.SemaphoreType.DMA,
        ],
    )
    def k(feat_hbm, lab_hbm, cent_hbm, out_hbm, idx_v, feat_v, rows_v,
          acc_v, gsem, fsem):
        wid = lax.axis_index("s") * nc + lax.axis_index("c")
        base = wid * bpw

        pltpu.sync_copy(lab_hbm.at[pl.ds(base, bpw)], idx_v)

        def issue(c, carry):
            labs = idx_v[pl.ds(c * _L, _L)]
            for j in range(_L):
                pltpu.async_copy(
                    cent_hbm.at[pl.ds(labs[j], 1)],
                    rows_v.at[pl.ds(c * _L + j, 1)],
                    gsem,
                )
            return carry

        lax.fori_loop(0, bpw // _L, issue, 0)

        def drain(i, carry):
            pltpu.make_async_copy(
                cent_hbm.at[pl.ds(0, 1)], rows_v.at[pl.ds(i, 1)], gsem
            ).wait()
            return carry

        acc = jnp.zeros((_L,), jnp.float32)
        for h in range(bpw // fch):
            pltpu.sync_copy(
                feat_hbm.at[pl.ds(base + h * fch, fch)], feat_v)
            if h == 0:
                lax.fori_loop(0, bpw, drain, 0)

            def body(i, a, _h=h):
                for j in range(_D // _L):
                    f = feat_v[i, pl.ds(j * _L, _L)]
                    cvec = rows_v[_h * fch + i, pl.ds(j * _L, _L)]
                    d = f - cvec
                    a = a + d * d
                return a

            acc = lax.fori_loop(0, fch, body, acc)

        acc_v[...] = acc
        pltpu.sync_copy(acc_v, out_hbm.at[pl.ds(wid * _L, _L)])

    return k


_K = _build()


@jax.jit
def kernel(features, labels, centers):
    partials = _K(features, labels.astype(jnp.int32), centers)
    return jnp.sum(partials) / features.shape[0]


# speedup vs baseline: 1.3821x; 1.0560x over previous
import functools

import jax
import jax.numpy as jnp
from jax import lax
from jax.experimental import pallas as pl
from jax.experimental.pallas import tpu as pltpu
from jax.experimental.pallas import tpu_sc as plsc

_B = 16384
_D = 64
_L = 16


def _build():
    info = plsc.get_sparse_core_info()
    nc, ns = info.num_cores, info.num_subcores
    nw = nc * ns
    bpw = _B // nw
    nq = 4
    qr = bpw // nq

    mesh = plsc.VectorSubcoreMesh(core_axis_name="c", subcore_axis_name="s")

    @functools.partial(
        pl.kernel,
        mesh=mesh,
        out_type=jax.ShapeDtypeStruct((nw * _L,), jnp.float32),
        scratch_types=[
            pltpu.VMEM((bpw,), jnp.int32),
            pltpu.VMEM((2, qr, _D), jnp.float32),
            pltpu.VMEM((bpw, _D), jnp.float32),
            pltpu.VMEM((_L,), jnp.float32),
            pltpu.SemaphoreType.DMA,
            pltpu.SemaphoreType.DMA,
            pltpu.SemaphoreType.DMA,
            pltpu.SemaphoreType.DMA,
            pltpu.SemaphoreType.DMA,
            pltpu.SemaphoreType.DMA,
        ],
    )
    def k(feat_hbm, lab_hbm, cent_hbm, out_hbm, idx_v, feat_v, rows_v,
          acc_v, fs0, fs1, g0, g1, g2, g3):
        fsems = (fs0, fs1)
        gsems = (g0, g1, g2, g3)
        wid = lax.axis_index("s") * nc + lax.axis_index("c")
        base = wid * bpw

        pltpu.sync_copy(lab_hbm.at[pl.ds(base, bpw)], idx_v)

        def feat_cp(q):
            return pltpu.async_copy(
                feat_hbm.at[pl.ds(base + q * qr, qr)],
                feat_v.at[q % 2], fsems[q % 2])

        feat_cp(0)
        feat_cp(1)

        for q in range(nq):
            def issue(g, carry, _q=q):
                c = _q * (qr // _L) + g
                labs = idx_v[pl.ds(c * _L, _L)]
                for j in range(_L):
                    pltpu.async_copy(
                        cent_hbm.at[pl.ds(labs[j], 1)],
                        rows_v.at[pl.ds(c * _L + j, 1)],
                        gsems[_q],
                    )
                return carry

            lax.fori_loop(0, qr // _L, issue, 0)

        acc = jnp.zeros((_L,), jnp.float32)
        for q in range(nq):
            pltpu.make_async_copy(
                feat_hbm.at[pl.ds(0, qr)], feat_v.at[q % 2], fsems[q % 2]
            ).wait()
            pltpu.make_async_copy(
                cent_hbm.at[pl.ds(0, qr)],
                rows_v.at[pl.ds(q * qr, qr)],
                gsems[q],
            ).wait()

            def body(i, acc, _q=q):
                r = _q * qr + i
                for j in range(_D // _L):
                    f = feat_v[_q % 2, i, pl.ds(j * _L, _L)]
                    cvec = rows_v[r, pl.ds(j * _L, _L)]
                    d = f - cvec
                    acc = acc + d * d
                return acc

            acc = lax.fori_loop(0, qr, body, acc)
            if q + 2 < nq:
                feat_cp(q + 2)

        acc_v[...] = acc
        pltpu.sync_copy(acc_v, out_hbm.at[pl.ds(wid * _L, _L)])

    return k


_K = _build()


@jax.jit
def kernel(features, labels, centers):
    partials = _K(features, labels.astype(jnp.int32), centers)
    return jnp.sum(partials) / features.shape[0]
